# baseline (speedup 1.0000x reference)
import jax
import jax.numpy as jnp
from jax.experimental import pallas as pl
from jax.experimental.pallas import tpu as pltpu

_H, _W = 320, 800
_THR = 0.1
_NCH = 4
_HB = _H // _NCH


def _outer(heat_hbm, off_hbm, err_hbm, out_hbm, hbuf, ofbuf, erbuf, obuf,
           insem, outsem):
    in_cps = []
    for i in range(_NCH):
        sl = pl.ds(i * _HB, _HB)
        cps = (
            pltpu.make_async_copy(heat_hbm.at[sl], hbuf.at[sl], insem.at[i]),
            pltpu.make_async_copy(off_hbm.at[:, sl], ofbuf.at[:, sl], insem.at[i]),
            pltpu.make_async_copy(err_hbm.at[:, sl], erbuf.at[:, sl], insem.at[i]),
        )
        in_cps.append(cps)
    for i in range(2):
        for cp in in_cps[i]:
            cp.start()

    out_cps = []
    for i in range(_NCH):
        sl = pl.ds(i * _HB, _HB)
        for cp in in_cps[i]:
            cp.wait()

        h = hbuf[sl]
        ninf = jnp.full((_HB, 1), -jnp.inf, dtype=jnp.float32)
        lft = jnp.concatenate([ninf, h[:, :-1]], axis=1)
        rgt = jnp.concatenate([h[:, 1:], ninf], axis=1)
        hmax = jnp.maximum(jnp.maximum(lft, rgt), h)
        nms = jnp.where(hmax == h, h, 0.0)
        m = nms > _THR

        xs = jax.lax.broadcasted_iota(jnp.int32, (_HB, _W), 1).astype(jnp.float32)
        ys = float(i * _HB) + jax.lax.broadcasted_iota(
            jnp.int32, (_HB, _W), 0
        ).astype(jnp.float32)

        obuf[0, sl] = nms
        obuf[1, sl] = jnp.where(m, xs + ofbuf[0, sl], 0.0)
        obuf[2, sl] = jnp.where(m, ys + ofbuf[1, sl], 0.0)
        obuf[3, sl] = jnp.where(m, xs + erbuf[0, sl], 0.0)
        obuf[4, sl] = jnp.where(m, ys + erbuf[1, sl], 0.0)

        cp = pltpu.make_async_copy(
            obuf.at[:, sl], out_hbm.at[:, sl], outsem.at[i]
        )
        cp.start()
        out_cps.append(cp)
        if i + 2 < _NCH:
            for nxt in in_cps[i + 2]:
                nxt.start()

    for cp in out_cps:
        cp.wait()


@jax.jit
def _decode(heat2d, off, err):
    return pl.pallas_call(
        _outer,
        in_specs=[pl.BlockSpec(memory_space=pl.ANY)] * 3,
        out_specs=pl.BlockSpec(memory_space=pl.ANY),
        out_shape=jax.ShapeDtypeStruct((5, _H, _W), jnp.float32),
        scratch_shapes=[
            pltpu.VMEM((_H, _W), jnp.float32),
            pltpu.VMEM((2, _H, _W), jnp.float32),
            pltpu.VMEM((2, _H, _W), jnp.float32),
            pltpu.VMEM((5, _H, _W), jnp.float32),
            pltpu.SemaphoreType.DMA((_NCH,)),
            pltpu.SemaphoreType.DMA((_NCH,)),
        ],
    )(heat2d, off, err)


def kernel(heat, offset, error):
    hf = heat.reshape(_H, _W)
    off = offset.reshape(2, _H, _W)
    err = error.reshape(2, _H, _W)
    out5 = _decode(hf, off, err)
    return jnp.transpose(out5, (1, 2, 0))


# speedup vs baseline: 1.3287x; 1.3287x over previous
import jax
import jax.numpy as jnp
from jax.experimental import pallas as pl
from jax.experimental.pallas import tpu as pltpu

_H, _W = 320, 800
_THR = 0.1
_NCH = 4
_HB = _H // _NCH


def _outer(heat_hbm, off_hbm, err_hbm, out_hbm, hbuf, ofbuf, erbuf, obuf,
           insem, outsem):
    in_cps = []
    for i in range(_NCH):
        sl = pl.ds(i * _HB, _HB)
        cps = (
            pltpu.make_async_copy(heat_hbm.at[sl], hbuf.at[sl], insem.at[i]),
            pltpu.make_async_copy(off_hbm.at[:, sl], ofbuf.at[:, sl], insem.at[i]),
            pltpu.make_async_copy(err_hbm.at[:, sl], erbuf.at[:, sl], insem.at[i]),
        )
        for cp in cps:
            cp.start()
        in_cps.append(cps)

    out_cps = []
    for j in range(2 * _NCH):
        i = j // 2
        sl = pl.ds(j * (_HB // 2), _HB // 2)
        if j % 2 == 0:
            for cp in in_cps[i]:
                cp.wait()

        hh = _HB // 2
        h = hbuf[sl]
        ninf = jnp.full((hh, 1), -jnp.inf, dtype=jnp.float32)
        lft = jnp.concatenate([ninf, h[:, :-1]], axis=1)
        rgt = jnp.concatenate([h[:, 1:], ninf], axis=1)
        hmax = jnp.maximum(jnp.maximum(lft, rgt), h)
        nms = jnp.where(hmax == h, h, 0.0)
        m = nms > _THR

        xs = jax.lax.broadcasted_iota(jnp.int32, (hh, _W), 1).astype(jnp.float32)
        ys = float(j * hh) + jax.lax.broadcasted_iota(
            jnp.int32, (hh, _W), 0
        ).astype(jnp.float32)

        obuf[0, sl] = nms
        obuf[1, sl] = jnp.where(m, xs + ofbuf[0, sl], 0.0)
        obuf[2, sl] = jnp.where(m, ys + ofbuf[1, sl], 0.0)
        obuf[3, sl] = jnp.where(m, xs + erbuf[0, sl], 0.0)
        obuf[4, sl] = jnp.where(m, ys + erbuf[1, sl], 0.0)

        cp = pltpu.make_async_copy(
            obuf.at[:, sl], out_hbm.at[:, sl], outsem.at[i]
        )
        cp.start()
        out_cps.append(cp)

    for cp in out_cps:
        cp.wait()


@jax.jit
def _decode(heat2d, off, err):
    return pl.pallas_call(
        _outer,
        in_specs=[pl.BlockSpec(memory_space=pl.ANY)] * 3,
        out_specs=pl.BlockSpec(memory_space=pl.ANY),
        out_shape=jax.ShapeDtypeStruct((5, _H, _W), jnp.float32),
        scratch_shapes=[
            pltpu.VMEM((_H, _W), jnp.float32),
            pltpu.VMEM((2, _H, _W), jnp.float32),
            pltpu.VMEM((2, _H, _W), jnp.float32),
            pltpu.VMEM((5, _H, _W), jnp.float32),
            pltpu.SemaphoreType.DMA((_NCH,)),
            pltpu.SemaphoreType.DMA((_NCH,)),
        ],
    )(heat2d, off, err)


def kernel(heat, offset, error):
    hf = heat.reshape(_H, _W)
    off = offset.reshape(2, _H, _W)
    err = error.reshape(2, _H, _W)
    out5 = _decode(hf, off, err)
    return jnp.transpose(out5, (1, 2, 0))


# speedup vs baseline: 1.3601x; 1.0236x over previous
import jax
import jax.numpy as jnp
from jax.experimental import pallas as pl
from jax.experimental.pallas import tpu as pltpu

_H, _W = 320, 800
_THR = 0.1
_NCH = 4
_HB = _H // _NCH


def _outer(heat_hbm, off_hbm, err_hbm, out_hbm, hbuf, ofbuf, erbuf, obuf,
           insem, outsem):
    in_cps = []
    for i in range(_NCH):
        sl = pl.ds(i * _HB, _HB)
        cps = (
            pltpu.make_async_copy(heat_hbm.at[sl], hbuf.at[sl], insem.at[i]),
            pltpu.make_async_copy(off_hbm.at[:, sl], ofbuf.at[:, sl], insem.at[i]),
            pltpu.make_async_copy(err_hbm.at[:, sl], erbuf.at[:, sl], insem.at[i]),
        )
        for cp in cps:
            cp.start()
        in_cps.append(cps)

    out_cps = []
    for i in range(_NCH):
        sl = pl.ds(i * _HB, _HB)
        for cp in in_cps[i]:
            cp.wait()

        h = hbuf[sl]
        ninf = jnp.full((_HB, 1), -jnp.inf, dtype=jnp.float32)
        lft = jnp.concatenate([ninf, h[:, :-1]], axis=1)
        rgt = jnp.concatenate([h[:, 1:], ninf], axis=1)
        hmax = jnp.maximum(jnp.maximum(lft, rgt), h)
        nms = jnp.where(hmax == h, h, 0.0)
        m = nms > _THR

        xs = jax.lax.broadcasted_iota(jnp.int32, (_HB, _W), 1).astype(jnp.float32)
        ys = float(i * _HB) + jax.lax.broadcasted_iota(
            jnp.int32, (_HB, _W), 0
        ).astype(jnp.float32)

        obuf[0, sl] = nms
        obuf[1, sl] = jnp.where(m, xs + ofbuf[0, sl], 0.0)
        obuf[2, sl] = jnp.where(m, ys + ofbuf[1, sl], 0.0)
        obuf[3, sl] = jnp.where(m, xs + erbuf[0, sl], 0.0)
        obuf[4, sl] = jnp.where(m, ys + erbuf[1, sl], 0.0)

        cp = pltpu.make_async_copy(
            obuf.at[:, sl], out_hbm.at[:, sl], outsem.at[i]
        )
        cp.start()
        out_cps.append(cp)

    for cp in out_cps:
        cp.wait()


@jax.jit
def _decode(heat2d, off, err):
    return pl.pallas_call(
        _outer,
        in_specs=[pl.BlockSpec(memory_space=pl.ANY)] * 3,
        out_specs=pl.BlockSpec(memory_space=pl.ANY),
        out_shape=jax.ShapeDtypeStruct((5, _H, _W), jnp.float32),
        scratch_shapes=[
            pltpu.VMEM((_H, _W), jnp.float32),
            pltpu.VMEM((2, _H, _W), jnp.float32),
            pltpu.VMEM((2, _H, _W), jnp.float32),
            pltpu.VMEM((5, _H, _W), jnp.float32),
            pltpu.SemaphoreType.DMA((_NCH,)),
            pltpu.SemaphoreType.DMA((_NCH,)),
        ],
    )(heat2d, off, err)


def kernel(heat, offset, error):
    hf = heat.reshape(_H, _W)
    off = offset.reshape(2, _H, _W)
    err = error.reshape(2, _H, _W)
    out5 = _decode(hf, off, err)
    return jnp.transpose(out5, (1, 2, 0))


# speedup vs baseline: 1.3656x; 1.0041x over previous
import jax
import jax.numpy as jnp
from jax.experimental import pallas as pl
from jax.experimental.pallas import tpu as pltpu

_H, _W = 320, 800
_THR = 0.1
_NCH = 4
_HB = _H // _NCH


def _outer(heat_hbm, off_hbm, err_hbm, out_hbm, hbuf, ofbuf, erbuf, obuf,
           insem, outsem):
    in_cps = []
    for i in range(_NCH):
        sl = pl.ds(i * _HB, _HB)
        cps = (
            pltpu.make_async_copy(heat_hbm.at[sl], hbuf.at[sl], insem.at[i]),
            pltpu.make_async_copy(off_hbm.at[:, sl], ofbuf.at[:, sl], insem.at[i]),
            pltpu.make_async_copy(err_hbm.at[:, sl], erbuf.at[:, sl], insem.at[i]),
        )
        for cp in cps:
            cp.start()
        in_cps.append(cps)

    out_cps = []
    for i in range(_NCH):
        sl = pl.ds(i * _HB, _HB)
        for cp in in_cps[i]:
            cp.wait()

        h = hbuf[sl]
        ninf = jnp.full((_HB, 1), -jnp.inf, dtype=jnp.float32)
        lft = jnp.concatenate([ninf, h[:, :-1]], axis=1)
        rgt = jnp.concatenate([h[:, 1:], ninf], axis=1)
        hmax = jnp.maximum(jnp.maximum(lft, rgt), h)
        nms = jnp.where(hmax == h, h, 0.0)
        m = nms > _THR

        xs = jax.lax.broadcasted_iota(jnp.int32, (_HB, _W), 1).astype(jnp.float32)
        ys = float(i * _HB) + jax.lax.broadcasted_iota(
            jnp.int32, (_HB, _W), 0
        ).astype(jnp.float32)

        obuf[0, sl] = nms
        obuf[1, sl] = jnp.where(m, xs + ofbuf[0, sl], 0.0)
        obuf[2, sl] = jnp.where(m, ys + ofbuf[1, sl], 0.0)
        obuf[3, sl] = jnp.where(m, xs + erbuf[0, sl], 0.0)
        obuf[4, sl] = jnp.where(m, ys + erbuf[1, sl], 0.0)

        for c in range(5):
            cp = pltpu.make_async_copy(
                obuf.at[c, sl], out_hbm.at[c, sl], outsem.at[i]
            )
            cp.start()
            out_cps.append(cp)

    for cp in out_cps:
        cp.wait()


@jax.jit
def _decode(heat2d, off, err):
    return pl.pallas_call(
        _outer,
        in_specs=[pl.BlockSpec(memory_space=pl.ANY)] * 3,
        out_specs=pl.BlockSpec(memory_space=pl.ANY),
        out_shape=jax.ShapeDtypeStruct((5, _H, _W), jnp.float32),
        scratch_shapes=[
            pltpu.VMEM((_H, _W), jnp.float32),
            pltpu.VMEM((2, _H, _W), jnp.float32),
            pltpu.VMEM((2, _H, _W), jnp.float32),
            pltpu.VMEM((5, _H, _W), jnp.float32),
            pltpu.SemaphoreType.DMA((_NCH,)),
            pltpu.SemaphoreType.DMA((_NCH,)),
        ],
    )(heat2d, off, err)


def kernel(heat, offset, error):
    hf = heat.reshape(_H, _W)
    off = offset.reshape(2, _H, _W)
    err = error.reshape(2, _H, _W)
    out5 = _decode(hf, off, err)
    return jnp.transpose(out5, (1, 2, 0))


# speedup vs baseline: 1.3684x; 1.0021x over previous
import jax
import jax.numpy as jnp
from jax.experimental import pallas as pl
from jax.experimental.pallas import tpu as pltpu

_H, _W = 320, 800
_THR = 0.1
_NCH = 4
_HB = _H // _NCH


def _outer(heat_hbm, off_hbm, err_hbm, out_hbm, hbuf, ofbuf, erbuf, obuf,
           insem, outsem):
    in_cps = []
    for i in range(_NCH):
        sl = pl.ds(i * _HB, _HB)
        cps = (
            pltpu.make_async_copy(heat_hbm.at[sl], hbuf.at[sl], insem.at[i]),
            pltpu.make_async_copy(off_hbm.at[:, sl], ofbuf.at[:, sl], insem.at[i]),
            pltpu.make_async_copy(err_hbm.at[:, sl], erbuf.at[:, sl], insem.at[i]),
        )
        for cp in cps:
            cp.start()
        in_cps.append(cps)

    out_cps = []
    for i in range(_NCH):
        sl = pl.ds(i * _HB, _HB)
        for cp in in_cps[i]:
            cp.wait()

        h = hbuf[sl]
        ninf = jnp.full((_HB, 1), -jnp.inf, dtype=jnp.float32)
        lft = jnp.concatenate([ninf, h[:, :-1]], axis=1)
        rgt = jnp.concatenate([h[:, 1:], ninf], axis=1)
        hmax = jnp.maximum(jnp.maximum(lft, rgt), h)
        nms = jnp.where(hmax == h, h, 0.0)
        m = nms > _THR

        xs = jax.lax.broadcasted_iota(jnp.int32, (_HB, _W), 1).astype(jnp.float32)
        ys = float(i * _HB) + jax.lax.broadcasted_iota(
            jnp.int32, (_HB, _W), 0
        ).astype(jnp.float32)

        obuf[0, sl] = nms
        obuf[1, sl] = jnp.where(m, xs + ofbuf[0, sl], 0.0)
        obuf[2, sl] = jnp.where(m, ys + ofbuf[1, sl], 0.0)
        obuf[3, sl] = jnp.where(m, xs + erbuf[0, sl], 0.0)
        obuf[4, sl] = jnp.where(m, ys + erbuf[1, sl], 0.0)

        cp = pltpu.make_async_copy(
            obuf.at[:, sl], out_hbm.at[:, sl], outsem.at[i]
        )
        cp.start()
        out_cps.append(cp)

    for cp in out_cps:
        cp.wait()


@jax.jit
def _decode(heat2d, off, err):
    return pl.pallas_call(
        _outer,
        in_specs=[pl.BlockSpec(memory_space=pl.ANY)] * 3,
        out_specs=pl.BlockSpec(memory_space=pl.ANY),
        out_shape=jax.ShapeDtypeStruct((5, _H, _W), jnp.float32),
        scratch_shapes=[
            pltpu.VMEM((_H, _W), jnp.float32),
            pltpu.VMEM((2, _H, _W), jnp.float32),
            pltpu.VMEM((2, _H, _W), jnp.float32),
            pltpu.VMEM((5, _H, _W), jnp.float32),
            pltpu.SemaphoreType.DMA((_NCH,)),
            pltpu.SemaphoreType.DMA((_NCH,)),
        ],
    )(heat2d, off, err)


def kernel(heat, offset, error):
    hf = heat.reshape(_H, _W)
    off = offset.reshape(2, _H, _W)
    err = error.reshape(2, _H, _W)
    out5 = _decode(hf, off, err)
    return jnp.transpose(out5, (1, 2, 0))
